# baseline (speedup 1.0000x reference)
import functools

import jax
import jax.numpy as jnp
from jax import lax
from jax.experimental import pallas as pl
from jax.experimental.pallas import tpu as pltpu
from jax.experimental.pallas import tpu_sc as plsc

V = 100000
D = 128
H = 128
L = 200

NC = 2
NS = 16
NW = NC * NS
PER = 8
ACTIVE = L // PER


def _sc_body(idx_hbm, emb_hbm, out_hbm, idx_v, rows_v, acc_v, sem):
    wid = lax.axis_index("s") * NC + lax.axis_index("c")
    zero = jnp.zeros((16,), jnp.float32)
    for c in range(D // 16):
        acc_v[pl.ds(c * 16, 16)] = zero

    @pl.when(wid < ACTIVE)
    def _():
        base = wid * PER
        pltpu.sync_copy(idx_hbm.at[pl.ds(base, PER)], idx_v)
        pltpu.async_copy(emb_hbm.at[idx_v], rows_v, sem).wait()
        for c in range(D // 16):
            acc = rows_v[0, pl.ds(c * 16, 16)]
            for r in range(1, PER):
                acc = acc + rows_v[r, pl.ds(c * 16, 16)]
            acc_v[pl.ds(c * 16, 16)] = acc

    pltpu.sync_copy(acc_v, out_hbm.at[wid])


@functools.cache
def _sc_gather_sum():
    return pl.kernel(
        _sc_body,
        out_type=jax.ShapeDtypeStruct((NW, D), jnp.float32),
        mesh=plsc.VectorSubcoreMesh(
            core_axis_name="c", subcore_axis_name="s",
            num_cores=NC, num_subcores=NS,
        ),
        scratch_types=[
            pltpu.VMEM((PER,), jnp.int32),
            pltpu.VMEM((PER, D), jnp.float32),
            pltpu.VMEM((D,), jnp.float32),
            pltpu.SemaphoreType.DMA,
        ],
    )


BV = 10000
K = V // BV


def _tc_body(part_ref, w1_ref, b1_ref, w2_ref, b2_ref, out_ref, h_scr):
    k = pl.program_id(0)

    @pl.when(k == 0)
    def _():
        embeds = jnp.sum(part_ref[...], axis=0, keepdims=True)
        h = lax.dot_general(
            embeds, w1_ref[...], (((1,), (1,)), ((), ())),
            preferred_element_type=jnp.float32,
        ) + b1_ref[...]
        h_scr[...] = jnp.maximum(h, 0.0)

    logits = lax.dot_general(
        h_scr[...], w2_ref[...], (((1,), (1,)), ((), ())),
        preferred_element_type=jnp.float32,
    ) + b2_ref[0]
    out_ref[k] = logits

    @pl.when(k == K - 1)
    def _():
        full = out_ref[...]
        m = jnp.max(full)
        s = jnp.sum(jnp.exp(full - m))
        out_ref[...] = full - (m + jnp.log(s))


def _tc_mlp(partials, W1, b1r, W2, b2r):
    return pl.pallas_call(
        _tc_body,
        grid=(K,),
        in_specs=[
            pl.BlockSpec((NW, D), lambda k: (0, 0)),
            pl.BlockSpec((H, D), lambda k: (0, 0)),
            pl.BlockSpec((1, H), lambda k: (0, 0)),
            pl.BlockSpec((BV, H), lambda k: (k, 0)),
            pl.BlockSpec((1, 1, BV), lambda k: (k, 0, 0)),
        ],
        out_specs=pl.BlockSpec((K, 1, BV), lambda k: (0, 0, 0)),
        out_shape=jax.ShapeDtypeStruct((K, 1, BV), jnp.float32),
        scratch_shapes=[pltpu.VMEM((1, H), jnp.float32)],
        compiler_params=pltpu.CompilerParams(
            dimension_semantics=("arbitrary",),
        ),
    )(partials, W1, b1r, W2, b2r)


def kernel(inputs, emb, W1, b1, W2, b2):
    partials = _sc_gather_sum()(inputs.astype(jnp.int32), emb)
    out3 = _tc_mlp(
        partials, W1, b1.reshape(1, H), W2, b2.reshape(K, 1, BV)
    )
    return out3.reshape(1, V)


# speedup vs baseline: 1.0086x; 1.0086x over previous
import functools

import jax
import jax.numpy as jnp
from jax import lax
from jax.experimental import pallas as pl
from jax.experimental.pallas import tpu as pltpu
from jax.experimental.pallas import tpu_sc as plsc

V = 100000
D = 128
H = 128
L = 200

NC = 2
NS = 16
NW = NC * NS
PER = 8
ACTIVE = L // PER


def _sc_body(idx_hbm, emb_hbm, out_hbm, idx_v, rows_v, acc_v, sem):
    wid = lax.axis_index("s") * NC + lax.axis_index("c")
    zero = jnp.zeros((16,), jnp.float32)
    for c in range(D // 16):
        acc_v[pl.ds(c * 16, 16)] = zero

    @pl.when(wid < ACTIVE)
    def _():
        base = wid * PER
        pltpu.sync_copy(idx_hbm.at[pl.ds(base, PER)], idx_v)
        pltpu.async_copy(emb_hbm.at[idx_v], rows_v, sem).wait()
        for c in range(D // 16):
            acc = rows_v[0, pl.ds(c * 16, 16)]
            for r in range(1, PER):
                acc = acc + rows_v[r, pl.ds(c * 16, 16)]
            acc_v[pl.ds(c * 16, 16)] = acc

    pltpu.sync_copy(acc_v, out_hbm.at[wid])


@functools.cache
def _sc_gather_sum():
    return pl.kernel(
        _sc_body,
        out_type=jax.ShapeDtypeStruct((NW, D), jnp.float32),
        mesh=plsc.VectorSubcoreMesh(
            core_axis_name="c", subcore_axis_name="s",
            num_cores=NC, num_subcores=NS,
        ),
        scratch_types=[
            pltpu.VMEM((PER,), jnp.int32),
            pltpu.VMEM((PER, D), jnp.float32),
            pltpu.VMEM((D,), jnp.float32),
            pltpu.SemaphoreType.DMA,
        ],
    )


BV = 20000
K = V // BV


def _tc_body(part_ref, w1_ref, b1_ref, w2_ref, b2_ref, out_ref, h_scr):
    k = pl.program_id(0)

    @pl.when(k == 0)
    def _():
        embeds = jnp.sum(part_ref[...], axis=0, keepdims=True)
        h = lax.dot_general(
            embeds, w1_ref[...], (((1,), (1,)), ((), ())),
            preferred_element_type=jnp.float32,
        ) + b1_ref[...]
        h_scr[...] = jnp.maximum(h, 0.0)

    logits = lax.dot_general(
        h_scr[...], w2_ref[...], (((1,), (1,)), ((), ())),
        preferred_element_type=jnp.float32,
    ) + b2_ref[0]
    out_ref[k] = logits

    @pl.when(k == K - 1)
    def _():
        full = out_ref[...]
        m = jnp.max(full)
        s = jnp.sum(jnp.exp(full - m))
        out_ref[...] = full - (m + jnp.log(s))


def _tc_mlp(partials, W1, b1r, W2, b2r):
    return pl.pallas_call(
        _tc_body,
        grid=(K,),
        in_specs=[
            pl.BlockSpec((NW, D), lambda k: (0, 0)),
            pl.BlockSpec((H, D), lambda k: (0, 0)),
            pl.BlockSpec((1, H), lambda k: (0, 0)),
            pl.BlockSpec((BV, H), lambda k: (k, 0)),
            pl.BlockSpec((1, 1, BV), lambda k: (k, 0, 0)),
        ],
        out_specs=pl.BlockSpec((K, 1, BV), lambda k: (0, 0, 0)),
        out_shape=jax.ShapeDtypeStruct((K, 1, BV), jnp.float32),
        scratch_shapes=[pltpu.VMEM((1, H), jnp.float32)],
        compiler_params=pltpu.CompilerParams(
            dimension_semantics=("arbitrary",),
        ),
    )(partials, W1, b1r, W2, b2r)


def kernel(inputs, emb, W1, b1, W2, b2):
    partials = _sc_gather_sum()(inputs.astype(jnp.int32), emb)
    out3 = _tc_mlp(
        partials, W1, b1.reshape(1, H), W2, b2.reshape(K, 1, BV)
    )
    return out3.reshape(1, V)


# speedup vs baseline: 1.0339x; 1.0251x over previous
import functools

import jax
import jax.numpy as jnp
from jax import lax
from jax.experimental import pallas as pl
from jax.experimental.pallas import tpu as pltpu
from jax.experimental.pallas import tpu_sc as plsc

V = 100000
D = 128
H = 128
L = 200

NC = 2
NS = 16
NW = NC * NS
PER = 8
ACTIVE = L // PER


def _sc_body(idx_hbm, emb_hbm, out_hbm, idx_v, rows_v, acc_v, sem):
    wid = lax.axis_index("s") * NC + lax.axis_index("c")
    zero = jnp.zeros((16,), jnp.float32)
    for c in range(D // 16):
        acc_v[pl.ds(c * 16, 16)] = zero

    @pl.when(wid < ACTIVE)
    def _():
        base = wid * PER
        pltpu.sync_copy(idx_hbm.at[pl.ds(base, PER)], idx_v)
        pltpu.async_copy(emb_hbm.at[idx_v], rows_v, sem).wait()
        for c in range(D // 16):
            acc = rows_v[0, pl.ds(c * 16, 16)]
            for r in range(1, PER):
                acc = acc + rows_v[r, pl.ds(c * 16, 16)]
            acc_v[pl.ds(c * 16, 16)] = acc

    pltpu.sync_copy(acc_v, out_hbm.at[wid])


@functools.cache
def _sc_gather_sum():
    return pl.kernel(
        _sc_body,
        out_type=jax.ShapeDtypeStruct((NW, D), jnp.float32),
        mesh=plsc.VectorSubcoreMesh(
            core_axis_name="c", subcore_axis_name="s",
            num_cores=NC, num_subcores=NS,
        ),
        scratch_types=[
            pltpu.VMEM((PER,), jnp.int32),
            pltpu.VMEM((PER, D), jnp.float32),
            pltpu.VMEM((D,), jnp.float32),
            pltpu.SemaphoreType.DMA,
        ],
    )


BV = 10000
K = V // BV
KH = K // 2


def _tc_body(part_ref, w1_ref, b1_ref, w2a_ref, w2b_ref, b2a_ref, b2b_ref,
             out_ref, h_scr):
    k = pl.program_id(0)

    @pl.when(k == 0)
    def _():
        embeds = jnp.sum(part_ref[...], axis=0, keepdims=True)
        h = lax.dot_general(
            embeds, w1_ref[...], (((1,), (1,)), ((), ())),
            preferred_element_type=jnp.float32,
        ) + b1_ref[...]
        h_scr[...] = jnp.maximum(h, 0.0)

    out_ref[k] = lax.dot_general(
        h_scr[...], w2a_ref[...], (((1,), (1,)), ((), ())),
        preferred_element_type=jnp.float32,
    ) + b2a_ref[0]
    out_ref[k + KH] = lax.dot_general(
        h_scr[...], w2b_ref[...], (((1,), (1,)), ((), ())),
        preferred_element_type=jnp.float32,
    ) + b2b_ref[0]

    @pl.when(k == KH - 1)
    def _():
        full = out_ref[...]
        m = jnp.max(full)
        s = jnp.sum(jnp.exp(full - m))
        out_ref[...] = full - (m + jnp.log(s))


def _tc_mlp(partials, W1, b1r, W2, b2r):
    return pl.pallas_call(
        _tc_body,
        grid=(KH,),
        in_specs=[
            pl.BlockSpec((NW, D), lambda k: (0, 0)),
            pl.BlockSpec((H, D), lambda k: (0, 0)),
            pl.BlockSpec((1, H), lambda k: (0, 0)),
            pl.BlockSpec((BV, H), lambda k: (k, 0)),
            pl.BlockSpec((BV, H), lambda k: (k + KH, 0)),
            pl.BlockSpec((1, 1, BV), lambda k: (k, 0, 0)),
            pl.BlockSpec((1, 1, BV), lambda k: (k + KH, 0, 0)),
        ],
        out_specs=pl.BlockSpec((K, 1, BV), lambda k: (0, 0, 0)),
        out_shape=jax.ShapeDtypeStruct((K, 1, BV), jnp.float32),
        scratch_shapes=[pltpu.VMEM((1, H), jnp.float32)],
        compiler_params=pltpu.CompilerParams(
            dimension_semantics=("arbitrary",),
        ),
    )(partials, W1, b1r, W2, W2, b2r, b2r)


def kernel(inputs, emb, W1, b1, W2, b2):
    partials = _sc_gather_sum()(inputs.astype(jnp.int32), emb)
    out3 = _tc_mlp(
        partials, W1, b1.reshape(1, H), W2, b2.reshape(K, 1, BV)
    )
    return out3.reshape(1, V)


# speedup vs baseline: 1.3821x; 1.3368x over previous
import jax
import jax.numpy as jnp
from jax import lax
from jax.experimental import pallas as pl
from jax.experimental.pallas import tpu as pltpu

V = 100000
D = 128
H = 128
L = 200

BV = 4000
K = V // BV


def _tc_body(idx_ref, emb_ref, w1_ref, b1_ref, w2_ref, out_ref,
             h_scr, rows_scr, gsem):
    k = pl.program_id(0)

    @pl.when(k == 0)
    def _():
        cps = []
        for i in range(L):
            cp = pltpu.make_async_copy(
                emb_ref.at[pl.ds(idx_ref[i], 1)],
                rows_scr.at[pl.ds(i, 1)], gsem,
            )
            cp.start()
            cps.append(cp)
        for cp in cps:
            cp.wait()
        embeds = jnp.sum(rows_scr[...], axis=0, keepdims=True)
        h = lax.dot_general(
            embeds, w1_ref[...], (((1,), (1,)), ((), ())),
            preferred_element_type=jnp.float32,
        ) + b1_ref[...]
        h_scr[...] = jnp.maximum(h, 0.0)

    out_ref[...] = lax.dot_general(
        h_scr[...], w2_ref[...], (((1,), (1,)), ((), ())),
        preferred_element_type=jnp.float32,
    )[None]


def _tc_mlp(idx, emb, W1, b1r, W2):
    return pl.pallas_call(
        _tc_body,
        grid=(K,),
        in_specs=[
            pl.BlockSpec(memory_space=pltpu.SMEM),
            pl.BlockSpec(memory_space=pl.ANY),
            pl.BlockSpec((H, D), lambda k: (0, 0)),
            pl.BlockSpec((1, H), lambda k: (0, 0)),
            pl.BlockSpec((BV, H), lambda k: (k, 0)),
        ],
        out_specs=pl.BlockSpec((1, 1, BV), lambda k: (k, 0, 0)),
        out_shape=jax.ShapeDtypeStruct((K, 1, BV), jnp.float32),
        scratch_shapes=[
            pltpu.VMEM((1, H), jnp.float32),
            pltpu.VMEM((L, D), jnp.float32),
            pltpu.SemaphoreType.DMA,
        ],
        compiler_params=pltpu.CompilerParams(
            dimension_semantics=("arbitrary",),
        ),
    )(idx, emb, W1, b1r, W2)


def _merge_body(ltc_ref, b2_ref, out_ref):
    m = jnp.max(ltc_ref[0] + b2_ref[pl.ds(0, BV)][None])
    for j in range(1, K):
        bj = ltc_ref[j] + b2_ref[pl.ds(j * BV, BV)][None]
        m = jnp.maximum(m, jnp.max(bj))
    s = jnp.zeros((), jnp.float32)
    for j in range(K):
        bj = ltc_ref[j] + b2_ref[pl.ds(j * BV, BV)][None]
        s = s + jnp.sum(jnp.exp(bj - m))
    lse = m + jnp.log(s)
    for j in range(K):
        bj = ltc_ref[j] + b2_ref[pl.ds(j * BV, BV)][None]
        out_ref[0:1, pl.ds(j * BV, BV)] = bj - lse


def _merge(ltc, b2):
    return pl.pallas_call(
        _merge_body,
        out_shape=jax.ShapeDtypeStruct((1, V), jnp.float32),
    )(ltc, b2)


def kernel(inputs, emb, W1, b1, W2, b2):
    idx = inputs.astype(jnp.int32)
    ltc = _tc_mlp(idx, emb, W1, b1.reshape(1, H), W2)
    return _merge(ltc, b2)


# speedup vs baseline: 1.8918x; 1.3688x over previous
import jax
import jax.numpy as jnp
from jax import lax
from jax.experimental import pallas as pl
from jax.experimental.pallas import tpu as pltpu

V = 100000
D = 128
H = 128
L = 200

BV = 10000
K = V // BV


def _tc_body(idx_ref, emb_ref, w1_ref, b1_ref, w2_ref, out_ref,
             h_scr, rows_scr, gsem):
    k = pl.program_id(0)

    @pl.when(k == 0)
    def _():
        cps = []
        for i in range(L):
            cp = pltpu.make_async_copy(
                emb_ref.at[pl.ds(idx_ref[i], 1)],
                rows_scr.at[pl.ds(i, 1)], gsem,
            )
            cp.start()
            cps.append(cp)
        for cp in cps:
            cp.wait()
        embeds = jnp.sum(rows_scr[...], axis=0, keepdims=True)
        h = lax.dot_general(
            embeds, w1_ref[...], (((1,), (1,)), ((), ())),
            preferred_element_type=jnp.float32,
        ) + b1_ref[...]
        h_scr[...] = jnp.maximum(h, 0.0)

    out_ref[...] = lax.dot_general(
        h_scr[...], w2_ref[...], (((1,), (1,)), ((), ())),
        preferred_element_type=jnp.float32,
    )[None]


def _tc_mlp(idx, emb, W1, b1r, W2):
    return pl.pallas_call(
        _tc_body,
        grid=(K,),
        in_specs=[
            pl.BlockSpec(memory_space=pltpu.SMEM),
            pl.BlockSpec(memory_space=pl.ANY),
            pl.BlockSpec((H, D), lambda k: (0, 0)),
            pl.BlockSpec((1, H), lambda k: (0, 0)),
            pl.BlockSpec((BV, H), lambda k: (k, 0)),
        ],
        out_specs=pl.BlockSpec((1, 1, BV), lambda k: (k, 0, 0)),
        out_shape=jax.ShapeDtypeStruct((K, 1, BV), jnp.float32),
        scratch_shapes=[
            pltpu.VMEM((1, H), jnp.float32),
            pltpu.VMEM((L, D), jnp.float32),
            pltpu.SemaphoreType.DMA,
        ],
        compiler_params=pltpu.CompilerParams(
            dimension_semantics=("arbitrary",),
        ),
    )(idx, emb, W1, b1r, W2)


def _merge_body(ltc_ref, b2_ref, out_ref):
    m = jnp.max(ltc_ref[0] + b2_ref[pl.ds(0, BV)][None])
    for j in range(1, K):
        bj = ltc_ref[j] + b2_ref[pl.ds(j * BV, BV)][None]
        m = jnp.maximum(m, jnp.max(bj))
    s = jnp.zeros((), jnp.float32)
    for j in range(K):
        bj = ltc_ref[j] + b2_ref[pl.ds(j * BV, BV)][None]
        s = s + jnp.sum(jnp.exp(bj - m))
    lse = m + jnp.log(s)
    for j in range(K):
        bj = ltc_ref[j] + b2_ref[pl.ds(j * BV, BV)][None]
        out_ref[0:1, pl.ds(j * BV, BV)] = bj - lse


def _merge(ltc, b2):
    return pl.pallas_call(
        _merge_body,
        out_shape=jax.ShapeDtypeStruct((1, V), jnp.float32),
    )(ltc, b2)


def kernel(inputs, emb, W1, b1, W2, b2):
    idx = inputs.astype(jnp.int32)
    ltc = _tc_mlp(idx, emb, W1, b1.reshape(1, H), W2)
    return _merge(ltc, b2)


# speedup vs baseline: 1.9252x; 1.0177x over previous
import jax
import jax.numpy as jnp
from jax import lax
from jax.experimental import pallas as pl
from jax.experimental.pallas import tpu as pltpu

V = 100000
D = 128
H = 128
L = 200

BV = 20000
K = V // BV


def _tc_body(idx_ref, emb_ref, w1_ref, b1_ref, w2_ref, out_ref,
             h_scr, rows_scr, gsem):
    k = pl.program_id(0)

    @pl.when(k == 0)
    def _():
        cps = []
        for i in range(L):
            cp = pltpu.make_async_copy(
                emb_ref.at[pl.ds(idx_ref[i], 1)],
                rows_scr.at[pl.ds(i, 1)], gsem,
            )
            cp.start()
            cps.append(cp)
        for cp in cps:
            cp.wait()
        embeds = jnp.sum(rows_scr[...], axis=0, keepdims=True)
        h = lax.dot_general(
            embeds, w1_ref[...], (((1,), (1,)), ((), ())),
            preferred_element_type=jnp.float32,
        ) + b1_ref[...]
        h_scr[...] = jnp.maximum(h, 0.0)

    out_ref[...] = lax.dot_general(
        h_scr[...], w2_ref[...], (((1,), (1,)), ((), ())),
        preferred_element_type=jnp.float32,
    )[None]


def _tc_mlp(idx, emb, W1, b1r, W2):
    return pl.pallas_call(
        _tc_body,
        grid=(K,),
        in_specs=[
            pl.BlockSpec(memory_space=pltpu.SMEM),
            pl.BlockSpec(memory_space=pl.ANY),
            pl.BlockSpec((H, D), lambda k: (0, 0)),
            pl.BlockSpec((1, H), lambda k: (0, 0)),
            pl.BlockSpec((BV, H), lambda k: (k, 0)),
        ],
        out_specs=pl.BlockSpec((1, 1, BV), lambda k: (k, 0, 0)),
        out_shape=jax.ShapeDtypeStruct((K, 1, BV), jnp.float32),
        scratch_shapes=[
            pltpu.VMEM((1, H), jnp.float32),
            pltpu.VMEM((L, D), jnp.float32),
            pltpu.SemaphoreType.DMA,
        ],
        compiler_params=pltpu.CompilerParams(
            dimension_semantics=("arbitrary",),
        ),
    )(idx, emb, W1, b1r, W2)


def _merge_body(ltc_ref, b2_ref, out_ref):
    m = jnp.max(ltc_ref[0] + b2_ref[pl.ds(0, BV)][None])
    for j in range(1, K):
        bj = ltc_ref[j] + b2_ref[pl.ds(j * BV, BV)][None]
        m = jnp.maximum(m, jnp.max(bj))
    s = jnp.zeros((), jnp.float32)
    for j in range(K):
        bj = ltc_ref[j] + b2_ref[pl.ds(j * BV, BV)][None]
        s = s + jnp.sum(jnp.exp(bj - m))
    lse = m + jnp.log(s)
    for j in range(K):
        bj = ltc_ref[j] + b2_ref[pl.ds(j * BV, BV)][None]
        out_ref[0:1, pl.ds(j * BV, BV)] = bj - lse


def _merge(ltc, b2):
    return pl.pallas_call(
        _merge_body,
        out_shape=jax.ShapeDtypeStruct((1, V), jnp.float32),
    )(ltc, b2)


def kernel(inputs, emb, W1, b1, W2, b2):
    idx = inputs.astype(jnp.int32)
    ltc = _tc_mlp(idx, emb, W1, b1.reshape(1, H), W2)
    return _merge(ltc, b2)
